# baseline (speedup 1.0000x reference)
import functools

import jax
import jax.numpy as jnp
from jax import lax
from jax.experimental import pallas as pl
from jax.experimental.pallas import tpu as pltpu
from jax.experimental.pallas import tpu_sc as plsc

EMB = 32
NUM_CORES = 2
NUM_SUBCORES = 16
LANES = 16
NUM_WORKERS = NUM_CORES * NUM_SUBCORES
CHUNK = 256
NSEM = 8


def _mf_sc(user_id, game_id, user_table, game_table):
    batch = user_id.shape[0]
    bpw = batch // NUM_WORKERS
    mesh = plsc.VectorSubcoreMesh(core_axis_name="c", subcore_axis_name="s")

    @functools.partial(
        pl.kernel,
        mesh=mesh,
        out_type=jax.ShapeDtypeStruct((batch,), jnp.float32),
        compiler_params=pltpu.CompilerParams(
            needs_layout_passes=False, skip_device_barrier=True),
        scratch_types=[
            pltpu.VMEM((bpw,), jnp.int32),
            pltpu.VMEM((bpw,), jnp.int32),
            pltpu.VMEM((CHUNK, EMB), jnp.float32),
            pltpu.VMEM((CHUNK, EMB), jnp.float32),
            pltpu.VMEM((bpw,), jnp.float32),
        ] + [pltpu.SemaphoreType.DMA] * (2 * NSEM),
    )
    def mf_kernel(uid_hbm, gid_hbm, ut_hbm, gt_hbm, out_hbm,
                  uid_v, gid_v, u_v, g_v, o_v, *sems):
        sems_u = sems[:NSEM]
        sems_g = sems[NSEM:]
        wid = lax.axis_index("s") * NUM_CORES + lax.axis_index("c")
        base = wid * bpw
        pltpu.sync_copy(uid_hbm.at[pl.ds(base, bpw)], uid_v)
        pltpu.sync_copy(gid_hbm.at[pl.ds(base, bpw)], gid_v)

        lanes = lax.iota(jnp.int32, LANES)

        @pl.loop(0, bpw, step=CHUNK)
        def _(r0):
            copies = []
            for k0 in range(0, CHUNK, LANES):
                uvec = uid_v[pl.ds(r0 + k0, LANES)]
                gvec = gid_v[pl.ds(r0 + k0, LANES)]
                for j in range(LANES):
                    copies.append(pltpu.async_copy(
                        ut_hbm.at[pl.ds(uvec[j], 1)],
                        u_v.at[pl.ds(k0 + j, 1)], sems_u[j % NSEM]))
                    copies.append(pltpu.async_copy(
                        gt_hbm.at[pl.ds(gvec[j], 1)],
                        g_v.at[pl.ds(k0 + j, 1)], sems_g[j % NSEM]))
            for cp in copies:
                cp.wait()

            @pl.loop(0, CHUNK, step=LANES)
            def _(k0):
                rows = k0 + lanes
                acc = jnp.zeros((LANES,), jnp.float32)
                for j in range(EMB):
                    cols = jnp.full((LANES,), j, jnp.int32)
                    u_col = plsc.load_gather(u_v, [rows, cols])
                    g_col = plsc.load_gather(g_v, [rows, cols])
                    acc = acc + u_col * g_col
                o_v[pl.ds(r0 + k0, LANES)] = 10.0 / (1.0 + jnp.exp(-acc))

        pltpu.sync_copy(o_v, out_hbm.at[pl.ds(base, bpw)])

    return mf_kernel(user_id, game_id, user_table, game_table)


def kernel(user_id, game_id, user_table, game_table):
    user_id = user_id.astype(jnp.int32)
    game_id = game_id.astype(jnp.int32)
    return _mf_sc(user_id, game_id, user_table, game_table)


# speedup vs baseline: 2.5655x; 2.5655x over previous
import functools

import jax
import jax.numpy as jnp
from jax import lax
from jax.experimental import pallas as pl
from jax.experimental.pallas import tpu as pltpu
from jax.experimental.pallas import tpu_sc as plsc

EMB = 32
BLKW = 128
NUM_CORES = 2
NUM_SUBCORES = 16
LANES = 16
NUM_WORKERS = NUM_CORES * NUM_SUBCORES
HALF = 8


def _mf_sc(user_id, game_id, user_table_t, game_table_t):
    batch = user_id.shape[0]
    bpw = batch // NUM_WORKERS
    mesh = plsc.VectorSubcoreMesh(core_axis_name="c", subcore_axis_name="s")

    @functools.partial(
        pl.kernel,
        mesh=mesh,
        out_type=jax.ShapeDtypeStruct((batch,), jnp.float32),
        compiler_params=pltpu.CompilerParams(
            needs_layout_passes=False, skip_device_barrier=True),
        scratch_types=[
            pltpu.VMEM((bpw,), jnp.int32),
            pltpu.VMEM((bpw,), jnp.int32),
            pltpu.VMEM((2 * HALF, EMB, BLKW), jnp.float32),
            pltpu.VMEM((LANES,), jnp.int32),
            pltpu.VMEM((LANES,), jnp.int32),
            pltpu.VMEM((bpw,), jnp.float32),
            pltpu.SemaphoreType.DMA,
            pltpu.SemaphoreType.DMA,
        ],
    )
    def mf_kernel(uid_hbm, gid_hbm, ut_hbm, gt_hbm, out_hbm,
                  uid_v, gid_v, blk_v, uc_v, gc_v, o_v, sem_u, sem_g):
        wid = lax.axis_index("s") * NUM_CORES + lax.axis_index("c")
        base = wid * bpw
        pltpu.sync_copy(uid_hbm.at[pl.ds(base, bpw)], uid_v)
        pltpu.sync_copy(gid_hbm.at[pl.ds(base, bpw)], gid_v)

        lanes = lax.iota(jnp.int32, LANES)
        half_mask = lanes < HALF
        dup8 = lax.rem(lanes, jnp.int32(HALF))
        slots_u = dup8
        slots_g = dup8 + HALF

        @pl.loop(0, bpw, step=LANES)
        def _(k0):
            uvec = uid_v[pl.ds(k0, LANES)]
            gvec = gid_v[pl.ds(k0, LANES)]
            uc_v[...] = lax.bitwise_and(uvec, jnp.int32(BLKW - 1))
            gc_v[...] = lax.bitwise_and(gvec, jnp.int32(BLKW - 1))
            ublk = lax.shift_right_logical(uvec, 7)
            gblk = lax.shift_right_logical(gvec, 7)

            accs = []
            for half in range(2):
                copies = []
                for j in range(HALF):
                    e = half * HALF + j
                    copies.append(pltpu.async_copy(
                        ut_hbm.at[:, pl.ds(ublk[e] * BLKW, BLKW)],
                        blk_v.at[j], sem_u))
                    copies.append(pltpu.async_copy(
                        gt_hbm.at[:, pl.ds(gblk[e] * BLKW, BLKW)],
                        blk_v.at[HALF + j], sem_g))
                for cp in copies:
                    cp.wait()

                cidx = dup8 + half * HALF
                ucols = plsc.load_gather(uc_v, [cidx])
                gcols = plsc.load_gather(gc_v, [cidx])
                acc = jnp.zeros((LANES,), jnp.float32)
                for c in range(EMB):
                    c_b = jnp.full((LANES,), c, jnp.int32)
                    vu = plsc.load_gather(blk_v, [slots_u, c_b, ucols])
                    vg = plsc.load_gather(blk_v, [slots_g, c_b, gcols])
                    acc = acc + vu * vg
                accs.append(acc)

            out = jnp.where(half_mask, accs[0], accs[1])
            o_v[pl.ds(k0, LANES)] = 10.0 / (1.0 + jnp.exp(-out))

        pltpu.sync_copy(o_v, out_hbm.at[pl.ds(base, bpw)])

    return mf_kernel(user_id, game_id, user_table_t, game_table_t)


def kernel(user_id, game_id, user_table, game_table):
    user_id = user_id.astype(jnp.int32)
    game_id = game_id.astype(jnp.int32)
    return _mf_sc(user_id, game_id, user_table.T, game_table.T)


# speedup vs baseline: 2.6813x; 1.0451x over previous
import functools

import jax
import jax.numpy as jnp
from jax import lax
from jax.experimental import pallas as pl
from jax.experimental.pallas import tpu as pltpu
from jax.experimental.pallas import tpu_sc as plsc

EMB = 32
BLKW = 128
NUM_CORES = 2
NUM_SUBCORES = 16
LANES = 16
NUM_WORKERS = NUM_CORES * NUM_SUBCORES
HALF = 4


def _mf_sc(user_id, game_id, user_table_t, game_table_t):
    batch = user_id.shape[0]
    bpw = batch // NUM_WORKERS
    rounds = bpw // HALF
    mesh = plsc.VectorSubcoreMesh(core_axis_name="c", subcore_axis_name="s")

    @functools.partial(
        pl.kernel,
        mesh=mesh,
        out_type=jax.ShapeDtypeStruct((batch,), jnp.float32),
        compiler_params=pltpu.CompilerParams(
            needs_layout_passes=False, skip_device_barrier=True),
        scratch_types=[
            pltpu.VMEM((bpw,), jnp.int32),
            pltpu.VMEM((bpw,), jnp.int32),
            pltpu.VMEM((4 * HALF, EMB, BLKW), jnp.float32),
            pltpu.VMEM((bpw,), jnp.float32),
            pltpu.SemaphoreType.DMA,
            pltpu.SemaphoreType.DMA,
        ],
    )
    def mf_kernel(uid_hbm, gid_hbm, ut_hbm, gt_hbm, out_hbm,
                  uid_v, gid_v, blk_v, o_v, sem_a, sem_b):
        wid = lax.axis_index("s") * NUM_CORES + lax.axis_index("c")
        base = wid * bpw
        pltpu.sync_copy(uid_hbm.at[pl.ds(base, bpw)], uid_v)
        pltpu.sync_copy(gid_hbm.at[pl.ds(base, bpw)], gid_v)

        lanes = lax.iota(jnp.int32, LANES)
        dup4 = lax.bitwise_and(lanes, jnp.int32(HALF - 1))
        quad = lax.shift_right_logical(lanes, 2)
        sems = (sem_a, sem_b)

        def issue(g, set_i):
            b = 2 * HALF * set_i
            idu = plsc.load_gather(uid_v, [dup4 + g * HALF])
            idg = plsc.load_gather(gid_v, [dup4 + g * HALF])
            ublk = lax.shift_right_logical(idu, 7)
            gblk = lax.shift_right_logical(idg, 7)
            for j in range(HALF):
                pltpu.async_copy(
                    ut_hbm.at[:, pl.ds(ublk[j] * BLKW, BLKW)],
                    blk_v.at[b + j], sems[set_i])
                pltpu.async_copy(
                    gt_hbm.at[:, pl.ds(gblk[j] * BLKW, BLKW)],
                    blk_v.at[b + HALF + j], sems[set_i])

        def drain(set_i):
            b = 2 * HALF * set_i
            for j in range(2 * HALF):
                pltpu.make_async_copy(
                    ut_hbm.at[:, pl.ds(0, BLKW)],
                    blk_v.at[b + j], sems[set_i]).wait()

        issue(0, 0)
        issue(1, 1)

        @pl.loop(0, rounds, step=4)
        def _(r0):
            accs = []
            for s in range(4):
                set_i = s % 2
                b = 2 * HALF * set_i
                drain(set_i)
                g = r0 + s
                idu = plsc.load_gather(uid_v, [dup4 + g * HALF])
                idg = plsc.load_gather(gid_v, [dup4 + g * HALF])
                ucols = lax.bitwise_and(idu, jnp.int32(BLKW - 1))
                gcols = lax.bitwise_and(idg, jnp.int32(BLKW - 1))
                slots_u = dup4 + b
                slots_g = dup4 + b + HALF
                acc = jnp.zeros((LANES,), jnp.float32)
                for c in range(EMB):
                    c_b = jnp.full((LANES,), c, jnp.int32)
                    vu = plsc.load_gather(blk_v, [slots_u, c_b, ucols])
                    vg = plsc.load_gather(blk_v, [slots_g, c_b, gcols])
                    acc = acc + vu * vg
                accs.append(acc)

                @pl.when(r0 + s + 2 < rounds)
                def _():
                    issue(g + 2, set_i)

            out = jnp.where(
                quad == 0, accs[0],
                jnp.where(quad == 1, accs[1],
                          jnp.where(quad == 2, accs[2], accs[3])))
            o_v[pl.ds(r0 * HALF, LANES)] = 10.0 / (1.0 + jnp.exp(-out))

        pltpu.sync_copy(o_v, out_hbm.at[pl.ds(base, bpw)])

    return mf_kernel(user_id, game_id, user_table_t, game_table_t)


def kernel(user_id, game_id, user_table, game_table):
    user_id = user_id.astype(jnp.int32)
    game_id = game_id.astype(jnp.int32)
    return _mf_sc(user_id, game_id, user_table.T, game_table.T)
